# baseline (speedup 1.0000x reference)
import functools
import jax
import jax.numpy as jnp
from jax import lax
from jax.experimental import pallas as pl
from jax.experimental.pallas import tpu as pltpu
from jax.experimental.pallas import tpu_sc as plsc

E_DIM = 30
ROW_PAD = 32
BATCH = 16384
NW = 32
B_PER_W = BATCH // NW
GROUPS = B_PER_W // 16


def _rsqrt(x):
    i = plsc.bitcast(x, jnp.int32)
    i = 0x5F3759DF - lax.shift_right_arithmetic(i, 1)
    y = plsc.bitcast(i, jnp.float32)
    for _ in range(3):
        y = y * (1.5 - 0.5 * x * y * y)
    return y


@functools.partial(
    pl.kernel,
    mesh=plsc.VectorSubcoreMesh(core_axis_name="c", subcore_axis_name="s"),
    out_type=jax.ShapeDtypeStruct((BATCH,), jnp.float32),
    compiler_params=pltpu.CompilerParams(
        needs_layout_passes=False, use_tc_tiling_on_sc=False),
    scratch_types=[
        pltpu.VMEM((B_PER_W,), jnp.int32),
        pltpu.VMEM((B_PER_W,), jnp.int32),
        pltpu.VMEM((B_PER_W,), jnp.int32),
        pltpu.VMEM((B_PER_W,), jnp.int32),
        pltpu.VMEM((B_PER_W, 8), jnp.int32),
        pltpu.VMEM((B_PER_W, 8), jnp.int32),
        pltpu.VMEM((B_PER_W, 8), jnp.int32),
        pltpu.VMEM((B_PER_W, 8), jnp.int32),
        pltpu.VMEM((B_PER_W, 2, E_DIM), jnp.float32),
        pltpu.VMEM((B_PER_W, 2, E_DIM), jnp.float32),
        pltpu.VMEM((B_PER_W,), jnp.float32),
        pltpu.SemaphoreType.DMA,
    ],
)
def _sc_embed_dot(midx_hbm, lidx_hbm, wm_hbm, wl_hbm, out_hbm,
                  mi_v, li_v, mo_v, lo_v, mra_v, lra_v, mrb_v, lrb_v,
                  m_w, l_w, r_v, sem):
    wid = lax.axis_index("s") * 2 + lax.axis_index("c")
    base = wid * B_PER_W

    pltpu.sync_copy(midx_hbm.at[pl.ds(base, B_PER_W)], mi_v)
    pltpu.sync_copy(lidx_hbm.at[pl.ds(base, B_PER_W)], li_v)

    thirty = jnp.full((16,), E_DIM, jnp.int32)

    def build(idx_v, off_v, rowa_v, rowb_v):
        def g_body(g, carry):
            o = g * 16
            iv = idx_v[pl.ds(o, 16)]
            w = lax.shift_left(iv, 5)
            ra = lax.div(w, thirty)
            kvec = o + lax.broadcasted_iota(jnp.int32, (16,), 0)
            zcol = jnp.zeros((16,), jnp.int32)
            plsc.store_scatter(rowa_v, [kvec, zcol], ra)
            plsc.store_scatter(rowb_v, [kvec, zcol], ra + 1)
            off_v[pl.ds(o, 16)] = w - ra * E_DIM
            return carry

        lax.fori_loop(0, GROUPS, g_body, 0)

    build(mi_v, mo_v, mra_v, mrb_v)
    build(li_v, lo_v, lra_v, lrb_v)

    def issue(k, carry):
        pltpu.async_copy(
            wm_hbm.at[mra_v.at[k, pl.ds(0, 1)]], m_w.at[k, pl.ds(0, 1)], sem)
        pltpu.async_copy(
            wm_hbm.at[mrb_v.at[k, pl.ds(0, 1)]], m_w.at[k, pl.ds(1, 1)], sem)
        pltpu.async_copy(
            wl_hbm.at[lra_v.at[k, pl.ds(0, 1)]], l_w.at[k, pl.ds(0, 1)], sem)
        pltpu.async_copy(
            wl_hbm.at[lrb_v.at[k, pl.ds(0, 1)]], l_w.at[k, pl.ds(1, 1)], sem)
        return carry

    lax.fori_loop(0, B_PER_W, issue, 0)

    def drain(k, carry):
        pltpu.make_async_copy(
            wm_hbm.at[mra_v.at[0, pl.ds(0, 1)]], m_w.at[k, pl.ds(0, 1)],
            sem).wait()
        pltpu.make_async_copy(
            wm_hbm.at[mra_v.at[0, pl.ds(0, 1)]], m_w.at[k, pl.ds(1, 1)],
            sem).wait()
        pltpu.make_async_copy(
            wl_hbm.at[lra_v.at[0, pl.ds(0, 1)]], l_w.at[k, pl.ds(0, 1)],
            sem).wait()
        pltpu.make_async_copy(
            wl_hbm.at[lra_v.at[0, pl.ds(0, 1)]], l_w.at[k, pl.ds(1, 1)],
            sem).wait()
        return carry

    lax.fori_loop(0, B_PER_W, drain, 0)

    iota16 = lax.broadcasted_iota(jnp.int32, (16,), 0)
    zeros = jnp.zeros((16,), jnp.float32)
    ones_i = jnp.full((16,), 1, jnp.int32)
    zeros_i = jnp.zeros((16,), jnp.int32)

    def group_body(g, carry):
        o = g * 16
        kvec = o + iota16
        mo = mo_v[pl.ds(o, 16)]
        lo = lo_v[pl.ds(o, 16)]
        md = zeros
        mm = zeros
        ll = zeros
        for j in range(E_DIM):
            tm = mo + j
            gem = jnp.where(tm >= E_DIM, ones_i, zeros_i)
            mv = plsc.load_gather(m_w, [kvec, gem, tm - gem * E_DIM])
            tl = lo + j
            gel = jnp.where(tl >= E_DIM, ones_i, zeros_i)
            lv = plsc.load_gather(l_w, [kvec, gel, tl - gel * E_DIM])
            md = md + mv * lv
            mm = mm + mv * mv
            ll = ll + lv * lv
        r_v[pl.ds(o, 16)] = md * _rsqrt(mm * ll)
        return carry

    lax.fori_loop(0, GROUPS, group_body, 0)

    pltpu.sync_copy(r_v, out_hbm.at[pl.ds(base, B_PER_W)])


def kernel(movie_batch, link_batch, W_movies, W_links):
    return _sc_embed_dot(
        movie_batch.astype(jnp.int32),
        link_batch.astype(jnp.int32),
        W_movies,
        W_links,
    )
